# baseline (speedup 1.0000x reference)
import functools

import jax
import jax.numpy as jnp
from jax import lax
from jax.experimental import pallas as pl
from jax.experimental.pallas import tpu as pltpu

B, N, M = 4, 4096, 4096
MH = M // 2


def _chamfer_body(x1_ref, y1_ref, x2_ref, y2_ref, out_ref):
    total = jnp.float32(0.0)
    for b in range(B):
        x1 = x1_ref[b, 0, :].reshape(N, 1)
        y1 = y1_ref[b, 0, :].reshape(N, 1)
        a2 = x1 * x1 + y1 * y1

        ones = jnp.ones((N, 1), jnp.bfloat16)
        am = jnp.concatenate(
            [
                (x1.astype(jnp.bfloat16) * jnp.bfloat16(-2.0)),
                (y1.astype(jnp.bfloat16) * jnp.bfloat16(-2.0)),
                ones,
                ones,
                ones,
            ],
            axis=1,
        )

        rowmin = jnp.full((N,), jnp.inf, jnp.float32)
        colsum = jnp.float32(0.0)
        for h in range(M // MH):
            x2 = x2_ref[b, 0, h * MH:(h + 1) * MH].reshape(1, MH)
            y2 = y2_ref[b, 0, h * MH:(h + 1) * MH].reshape(1, MH)
            b2 = x2 * x2 + y2 * y2
            b2h1 = b2.astype(jnp.bfloat16)
            r1 = b2 - b2h1.astype(jnp.float32)
            b2h2 = r1.astype(jnp.bfloat16)
            b2h3 = (r1 - b2h2.astype(jnp.float32)).astype(jnp.bfloat16)
            bm = jnp.concatenate(
                [
                    x2.astype(jnp.bfloat16),
                    y2.astype(jnp.bfloat16),
                    b2h1,
                    b2h2,
                    b2h3,
                ],
                axis=0,
            )

            q = lax.dot_general(
                am, bm, (((1,), (0,)), ((), ())),
                preferred_element_type=jnp.float32,
            )
            s = q + a2

            rowmin = jnp.minimum(rowmin, jnp.min(s, axis=1))
            colsum = colsum + jnp.sum(
                jnp.maximum(jnp.min(s, axis=0), 0.0)
            )
        total = total + (
            jnp.sum(jnp.maximum(rowmin, 0.0)) * (1.0 / N)
            + colsum * (1.0 / M)
        )
    out_ref[0, 0] = total


@jax.jit
def kernel(points1, points2):
    x1 = points1[..., 0].reshape(B, 1, N)
    y1 = points1[..., 1].reshape(B, 1, N)
    x2 = points2[..., 0].reshape(B, 1, M)
    y2 = points2[..., 1].reshape(B, 1, M)

    out = pl.pallas_call(
        _chamfer_body,
        out_specs=pl.BlockSpec(memory_space=pltpu.SMEM),
        out_shape=jax.ShapeDtypeStruct((1, 1), jnp.float32),
    )(x1, y1, x2, y2)
    return out[0, 0]


# speedup vs baseline: 1.0118x; 1.0118x over previous
import functools

import jax
import jax.numpy as jnp
from jax import lax
from jax.experimental import pallas as pl
from jax.experimental.pallas import tpu as pltpu

B, N, M = 4, 4096, 4096


def _split3(v):
    h1 = v.astype(jnp.bfloat16)
    r1 = v - h1.astype(jnp.float32)
    h2 = r1.astype(jnp.bfloat16)
    h3 = (r1 - h2.astype(jnp.float32)).astype(jnp.bfloat16)
    return h1, h2, h3


def _chamfer_body(x1_ref, y1_ref, x2_ref, y2_ref, out_ref):
    b = pl.program_id(0)

    x1 = x1_ref[0, 0, :].reshape(N, 1)
    y1 = y1_ref[0, 0, :].reshape(N, 1)
    x2 = x2_ref[0, 0, :].reshape(1, M)
    y2 = y2_ref[0, 0, :].reshape(1, M)

    a2 = x1 * x1 + y1 * y1
    a2h1, a2h2, a2h3 = _split3(a2)
    ones_c = jnp.ones((N, 1), jnp.bfloat16)
    am = jnp.concatenate(
        [
            (x1.astype(jnp.bfloat16) * jnp.bfloat16(-2.0)),
            (y1.astype(jnp.bfloat16) * jnp.bfloat16(-2.0)),
            ones_c,
            ones_c,
            ones_c,
            a2h1,
            a2h2,
            a2h3,
        ],
        axis=1,
    )

    b2 = x2 * x2 + y2 * y2
    b2h1, b2h2, b2h3 = _split3(b2)
    ones_r = jnp.ones((1, M), jnp.bfloat16)
    bm = jnp.concatenate(
        [
            x2.astype(jnp.bfloat16),
            y2.astype(jnp.bfloat16),
            b2h1,
            b2h2,
            b2h3,
            ones_r,
            ones_r,
            ones_r,
        ],
        axis=0,
    )

    s = lax.dot_general(
        am, bm, (((1,), (0,)), ((), ())),
        preferred_element_type=jnp.float32,
    )

    rowmin = jnp.min(s, axis=1)
    colmin = jnp.min(s, axis=0)

    cost = (
        jnp.sum(jnp.maximum(rowmin, 0.0)) * (1.0 / N)
        + jnp.sum(jnp.maximum(colmin, 0.0)) * (1.0 / M)
    )

    @pl.when(b == 0)
    def _init():
        out_ref[0, 0] = cost

    @pl.when(b != 0)
    def _acc():
        out_ref[0, 0] += cost


@jax.jit
def kernel(points1, points2):
    x1 = points1[..., 0].reshape(B, 1, N)
    y1 = points1[..., 1].reshape(B, 1, N)
    x2 = points2[..., 0].reshape(B, 1, M)
    y2 = points2[..., 1].reshape(B, 1, M)

    out = pl.pallas_call(
        _chamfer_body,
        grid=(B,),
        in_specs=[
            pl.BlockSpec((1, 1, N), lambda b: (b, 0, 0)),
            pl.BlockSpec((1, 1, N), lambda b: (b, 0, 0)),
            pl.BlockSpec((1, 1, M), lambda b: (b, 0, 0)),
            pl.BlockSpec((1, 1, M), lambda b: (b, 0, 0)),
        ],
        out_specs=pl.BlockSpec(
            (1, 1), lambda b: (0, 0), memory_space=pltpu.SMEM
        ),
        out_shape=jax.ShapeDtypeStruct((1, 1), jnp.float32),
    )(x1, y1, x2, y2)
    return out[0, 0]
